# Initial kernel scaffold; baseline (speedup 1.0000x reference)
import jax
import jax.numpy as jnp
from jax.experimental import pallas as pl


def kernel(x_enc, x_mark_enc, x_dec, x_mark_dec, params):
    raise NotImplementedError("write your pallas kernel here")


# speedup vs baseline: 1.3963x; 1.3963x over previous
import jax
import jax.numpy as jnp
from jax.experimental import pallas as pl
from jax.experimental.pallas import tpu as pltpu

BATCH = 16
SEQ_LEN = 512
PRED_LEN = 96
ENC_IN = 128
D_MODEL = 512
D_FF = 1024
E_LAYERS = 3
PATCH_LEN = 16
N_PATCH = SEQ_LEN // PATCH_LEN
NUM_SHARED = 2
NUM_EXPERTS = 8
TOP_K = 2
BALANCE_COEFF = 0.01
EPS = 1e-5

_WNAMES = (
    ['aff_w', 'aff_b']
    + sum([['mpd_w_%d' % i, 'mpd_b_%d' % i, 'tib_wc_%d' % i, 'tib_bc_%d' % i,
            'tib_w1_%d' % i, 'tib_b1_%d' % i, 'tib_w2_%d' % i, 'tib_b2_%d' % i,
            'tib_g_%d' % i, 'tib_be_%d' % i] for i in range(E_LAYERS)], [])
    + sum([['g1_w_%d' % i, 'g1_b_%d' % i, 'g2_w_%d' % i, 'g2_b_%d' % i,
            'proj_w_%d' % i, 'proj_b_%d' % i] for i in range(E_LAYERS - 1)], [])
    + ['moe_wg', 'moe_we1', 'moe_be1', 'moe_we2', 'moe_be2',
       'moe_ws1', 'moe_bs1', 'moe_ws2', 'moe_bs2', 'moe_wr', 'moe_br',
       'res_w1', 'res_b1', 'res_w2', 'res_b2']
)

_BF = jnp.bfloat16


def _mm(a, b):
    return jax.lax.dot(a.astype(_BF), b.astype(_BF),
                       preferred_element_type=jnp.float32)


def _fwd_kernel(x_ref, *refs):
    w = {name: refs[i] for i, name in enumerate(_WNAMES)}
    out_ref, ms_ref, ps_ref = refs[len(_WNAMES):]

    xb = x_ref[0]
    mean_c = jnp.mean(xb, axis=0, keepdims=True)
    var_c = jnp.mean((xb - mean_c) ** 2, axis=0, keepdims=True)
    std_c = jnp.sqrt(var_c + EPS)
    xn = (xb - mean_c) / std_c
    xn = xn * w['aff_w'][0] + w['aff_b'][0]
    x = xn.T

    xr = _mm(jax.nn.gelu(_mm(x, w['res_w1'][...]) + w['res_b1'][0]),
             w['res_w2'][...]) + w['res_b2'][0]

    hsum = jnp.zeros((ENC_IN, D_MODEL), jnp.float32)
    prev = None
    for i in range(E_LAYERS):
        if prev is None:
            x_in = x
        else:
            g = jax.nn.gelu(_mm(prev, w['g1_w_%d' % (i - 1)][...])
                            + w['g1_b_%d' % (i - 1)][0])
            gpre = jax.lax.dot_general(
                w['g2_w_%d' % (i - 1)][...].astype(_BF), g.astype(_BF),
                (((0,), (1,)), ((), ())),
                preferred_element_type=jnp.float32).T
            gate = jax.nn.sigmoid(gpre + w['g2_b_%d' % (i - 1)][0])
            proj = _mm(prev, w['proj_w_%d' % (i - 1)][...]) + w['proj_b_%d' % (i - 1)][0]
            x_in = x + gate * proj
        t = _mm(x_in, w['mpd_w_%d' % i][...]) + w['mpd_b_%d' % i][0]
        u = jax.lax.dot_general(w['tib_wc_%d' % i][...].astype(_BF),
                                t.astype(_BF),
                                (((0,), (0,)), ((), ())),
                                preferred_element_type=jnp.float32)
        u = u + w['tib_bc_%d' % i][0].T[:, None]
        t = t + u
        h = _mm(jax.nn.gelu(_mm(t, w['tib_w1_%d' % i][...]) + w['tib_b1_%d' % i][0]),
                w['tib_w2_%d' % i][...]) + w['tib_b2_%d' % i][0]
        t = t + h
        mu = jnp.mean(t, axis=-1, keepdims=True)
        v = jnp.mean((t - mu) ** 2, axis=-1, keepdims=True)
        t = (t - mu) / jnp.sqrt(v + EPS) * w['tib_g_%d' % i][0] + w['tib_be_%d' % i][0]
        hsum = hsum + t
        prev = t

    hmoe = hsum * (1.0 / E_LAYERS)

    logits = jax.lax.dot_general(
        w['moe_wg'][...].astype(_BF), hmoe.astype(_BF),
        (((0,), (1,)), ((), ())),
        preferred_element_type=jnp.float32).T
    m = jnp.max(logits, axis=-1, keepdims=True)
    ex = jnp.exp(logits - m)
    probs = ex / jnp.sum(ex, axis=-1, keepdims=True)
    iota_e = jax.lax.broadcasted_iota(jnp.int32, (ENC_IN, NUM_EXPERTS), 1)
    m1 = jnp.max(probs, axis=-1, keepdims=True)
    i1 = jnp.min(jnp.where(probs == m1, iota_e, NUM_EXPERTS), axis=-1, keepdims=True)
    sel1 = iota_e == i1
    pmasked = jnp.where(sel1, -jnp.inf, probs)
    m2 = jnp.max(pmasked, axis=-1, keepdims=True)
    i2 = jnp.min(jnp.where(pmasked == m2, iota_e, NUM_EXPERTS), axis=-1, keepdims=True)
    mask = (sel1 | (iota_e == i2)).astype(jnp.float32)
    gates = probs * mask
    gates = gates / (jnp.sum(gates, axis=-1, keepdims=True) + 1e-9)

    out = jnp.zeros((ENC_IN, PRED_LEN), jnp.float32)
    for e in range(NUM_EXPERTS):
        h1 = jax.nn.gelu(_mm(hmoe, w['moe_we1'][e]) + w['moe_be1'][e][None, :])
        eo = _mm(h1, w['moe_we2'][e]) + w['moe_be2'][e][None, :]
        out = out + gates[:, e][:, None] * eo
    for s in range(NUM_SHARED):
        hs = jax.nn.gelu(_mm(hmoe, w['moe_ws1'][s]) + w['moe_bs1'][s][None, :])
        so = _mm(hs, w['moe_ws2'][s]) + w['moe_bs2'][s][None, :]
        out = out + (1.0 / NUM_SHARED) * so
    out = out + _mm(hmoe, w['moe_wr'][...]) + w['moe_br'][0]

    total = out + xr
    total = (total - w['aff_b'][0].T[:, None]) / (w['aff_w'][0].T[:, None] + EPS)
    total = total * std_c.T + mean_c.T
    out_ref[0] = total.T

    ms_ref[0, 0] = jnp.sum(mask, axis=0)
    ps_ref[0, 0] = jnp.sum(probs, axis=0)


def kernel(x_enc, x_mark_enc, x_dec, x_mark_dec, params):
    del x_mark_enc, x_dec, x_mark_dec
    ws = []
    for name in _WNAMES:
        a = params[name]
        if name.startswith('mpd_w'):
            a = jnp.tile(a.astype(_BF), (N_PATCH, 1)) * _BF(1.0 / N_PATCH)
        elif a.ndim >= 2 and not name.startswith('moe_wg'):
            a = a.astype(_BF)
        if a.ndim == 1:
            a = a.reshape(1, -1)
        ws.append(a)

    in_specs = [pl.BlockSpec((1, SEQ_LEN, ENC_IN), lambda b: (b, 0, 0))]
    for a in ws:
        nd = a.ndim
        in_specs.append(pl.BlockSpec(a.shape, (lambda b, _n=nd: (0,) * _n)))

    out_shapes = (
        jax.ShapeDtypeStruct((BATCH, PRED_LEN, ENC_IN), jnp.float32),
        jax.ShapeDtypeStruct((BATCH, 1, NUM_EXPERTS), jnp.float32),
        jax.ShapeDtypeStruct((BATCH, 1, NUM_EXPERTS), jnp.float32),
    )
    out_specs = (
        pl.BlockSpec((1, PRED_LEN, ENC_IN), lambda b: (b, 0, 0)),
        pl.BlockSpec((1, 1, NUM_EXPERTS), lambda b: (b, 0, 0)),
        pl.BlockSpec((1, 1, NUM_EXPERTS), lambda b: (b, 0, 0)),
    )

    out, msum, psum = pl.pallas_call(
        _fwd_kernel,
        grid=(BATCH,),
        in_specs=in_specs,
        out_specs=out_specs,
        out_shape=out_shapes,
        compiler_params=pltpu.CompilerParams(
            dimension_semantics=("parallel",)),
    )(x_enc, *ws)

    f_frac = jnp.sum(msum, axis=(0, 1)) / (BATCH * ENC_IN)
    p_mean = jnp.sum(psum, axis=(0, 1)) / (BATCH * ENC_IN)
    balance_loss = BALANCE_COEFF * NUM_EXPERTS * jnp.sum(f_frac * p_mean)
    return out, balance_loss
